# baseline (speedup 1.0000x reference)
import functools

import jax
import jax.numpy as jnp
from jax import lax
from jax.experimental import pallas as pl
from jax.experimental.pallas import tpu as pltpu
from jax.experimental.pallas import tpu_sc as plsc

H = 1024
F = 3584
E = 8
TOPK = 2
T = 2048
S = T * TOPK
BT = 128
NB = S // BT + E
FT = 512
NF = F // FT

NC = 2
NS = 16
NW = NC * NS

_SC_MESH = plsc.VectorSubcoreMesh(core_axis_name="c", subcore_axis_name="s")


TB = 256


def _router_body(x_ref, gw_ref, logits_ref, idx_ref, w_ref):
    x = x_ref[...]
    gw = gw_ref[...]
    logits = lax.dot_general(x, gw, (((1,), (1,)), ((), ())),
                             preferred_element_type=jnp.float32)
    logits_ref[...] = logits
    lane = lax.broadcasted_iota(jnp.int32, (TB, E), 1)
    m1 = jnp.max(logits, axis=1, keepdims=True)
    i1 = jnp.min(jnp.where(logits == m1, lane, E), axis=1, keepdims=True)
    masked = jnp.where(lane == i1, -jnp.float32(1e30), logits)
    m2 = jnp.max(masked, axis=1, keepdims=True)
    i2 = jnp.min(jnp.where(masked == m2, lane, E), axis=1, keepdims=True)
    p1 = 1.0 / (1.0 + jnp.exp(m2 - m1))
    idx_ref[...] = jnp.concatenate([i1, i2], axis=1)
    w_ref[...] = jnp.concatenate([p1, 1.0 - p1], axis=1)


def _router(x, gate_w):
    return pl.pallas_call(
        _router_body,
        grid=(T // TB,),
        in_specs=[
            pl.BlockSpec((TB, H), lambda i: (i, 0)),
            pl.BlockSpec((E, H), lambda i: (0, 0)),
        ],
        out_specs=[
            pl.BlockSpec((TB, E), lambda i: (i, 0)),
            pl.BlockSpec((TB, TOPK), lambda i: (i, 0)),
            pl.BlockSpec((TB, TOPK), lambda i: (i, 0)),
        ],
        out_shape=[
            jax.ShapeDtypeStruct((T, E), jnp.float32),
            jax.ShapeDtypeStruct((T, TOPK), jnp.int32),
            jax.ShapeDtypeStruct((T, TOPK), jnp.float32),
        ],
    )(x, gate_w)


RPW = NB * BT // NW
DCH = 32


@functools.partial(
    pl.kernel,
    mesh=_SC_MESH,
    out_type=jax.ShapeDtypeStruct((NB * BT, H), jnp.float32),
    scratch_types=[
        pltpu.VMEM((DCH,), jnp.int32),
        pltpu.VMEM((DCH, H), jnp.float32),
        pltpu.SemaphoreType.DMA,
    ],
)
def _dispatch(x_hbm, idx_hbm, xs_hbm, idx_v, rows_v, sem):
    wid = lax.axis_index("s") * NC + lax.axis_index("c")
    base = wid * RPW

    def chunk(c, carry):
        off = base + c * DCH
        pltpu.sync_copy(idx_hbm.at[pl.ds(off, DCH)], idx_v)
        pltpu.async_copy(x_hbm.at[idx_v], rows_v, sem).wait()
        pltpu.sync_copy(rows_v, xs_hbm.at[pl.ds(off, DCH)])
        return carry

    lax.fori_loop(0, RPW // DCH, chunk, 0)


def _ffn1_body(seid_ref, sval_ref, xs_ref, ws_ref, w1_ref, w3_ref, hs_ref):
    b = pl.program_id(1)

    @pl.when(sval_ref[b] > 0)
    def _():
        x = xs_ref[pl.ds(b * BT, BT), :].astype(jnp.bfloat16)
        g = lax.dot_general(x, w1_ref[0], (((1,), (1,)), ((), ())),
                            preferred_element_type=jnp.float32)
        u = lax.dot_general(x, w3_ref[0], (((1,), (1,)), ((), ())),
                            preferred_element_type=jnp.float32)
        h = (g * jax.nn.sigmoid(g)) * (u * ws_ref[...])
        hs_ref[...] = h.astype(jnp.bfloat16)


def _ffn1(beid, bval, xs, ws, w1, w3):
    grid_spec = pltpu.PrefetchScalarGridSpec(
        num_scalar_prefetch=2,
        grid=(NF, NB),
        in_specs=[
            pl.BlockSpec((NB * BT, H), lambda f, b, seid, sval: (0, 0)),
            pl.BlockSpec((BT, 1), lambda f, b, seid, sval: (b, 0)),
            pl.BlockSpec((1, FT, H), lambda f, b, seid, sval: (seid[b], f, 0)),
            pl.BlockSpec((1, FT, H), lambda f, b, seid, sval: (seid[b], f, 0)),
        ],
        out_specs=pl.BlockSpec((BT, FT), lambda f, b, seid, sval: (b, f)),
    )
    return pl.pallas_call(
        _ffn1_body,
        grid_spec=grid_spec,
        out_shape=jax.ShapeDtypeStruct((NB * BT, F), jnp.bfloat16),
        compiler_params=pltpu.CompilerParams(
            dimension_semantics=("arbitrary", "arbitrary")),
    )(beid, bval, xs, ws, w1, w3)


def _ffn2_body(seid_ref, sval_ref, hs_ref, w2_ref, y_ref):
    @pl.when(sval_ref[pl.program_id(0)] > 0)
    def _():
        y_ref[...] = lax.dot_general(
            hs_ref[...], w2_ref[0], (((1,), (1,)), ((), ())),
            preferred_element_type=jnp.float32)


def _ffn2(beid, bval, hs, w2):
    grid_spec = pltpu.PrefetchScalarGridSpec(
        num_scalar_prefetch=2,
        grid=(NB,),
        in_specs=[
            pl.BlockSpec((BT, F), lambda b, seid, sval: (b, 0)),
            pl.BlockSpec((1, H, F), lambda b, seid, sval: (seid[b], 0, 0)),
        ],
        out_specs=pl.BlockSpec((BT, H), lambda b, seid, sval: (b, 0)),
    )
    return pl.pallas_call(
        _ffn2_body,
        grid_spec=grid_spec,
        out_shape=jax.ShapeDtypeStruct((NB * BT, H), jnp.float32),
        compiler_params=pltpu.CompilerParams(
            dimension_semantics=("arbitrary",)),
    )(beid, bval, hs, w2)


TPW = T // NW
CCH = 32


@functools.partial(
    pl.kernel,
    mesh=_SC_MESH,
    out_type=jax.ShapeDtypeStruct((T, H), jnp.float32),
    scratch_types=[
        pltpu.VMEM((CCH,), jnp.int32),
        pltpu.VMEM((CCH,), jnp.int32),
        pltpu.VMEM((CCH, H), jnp.float32),
        pltpu.VMEM((CCH, H), jnp.float32),
        pltpu.SemaphoreType.DMA,
        pltpu.SemaphoreType.DMA,
    ],
)
def _combine(y_hbm, pos0_hbm, pos1_hbm, out_hbm, i0_v, i1_v, b0_v, b1_v,
             sem0, sem1):
    wid = lax.axis_index("s") * NC + lax.axis_index("c")
    base = wid * TPW

    def chunk(c, carry):
        off = base + c * CCH
        pltpu.sync_copy(pos0_hbm.at[pl.ds(off, CCH)], i0_v)
        pltpu.sync_copy(pos1_hbm.at[pl.ds(off, CCH)], i1_v)
        cp0 = pltpu.async_copy(y_hbm.at[i0_v], b0_v, sem0)
        cp1 = pltpu.async_copy(y_hbm.at[i1_v], b1_v, sem1)
        cp0.wait()
        cp1.wait()

        def row(r, rc):
            def lanes(j, jc):
                sl = pl.ds(j * 16, 16)
                b0_v[r, sl] = b0_v[r, sl] + b1_v[r, sl]
                return jc
            return lax.fori_loop(0, H // 16, lanes, rc)

        lax.fori_loop(0, CCH, row, 0)
        pltpu.sync_copy(b0_v, out_hbm.at[pl.ds(off, CCH)])
        return carry

    lax.fori_loop(0, TPW // CCH, chunk, 0)


def kernel(hidden_states, gate_w, w1, w3, w2):
    orig_shape = hidden_states.shape
    x = hidden_states.reshape(T, H)

    logits, top_idx, top_w = _router(x, gate_w)

    eid = top_idx.reshape(S)
    wslot = top_w.reshape(S)
    tok = jnp.arange(S, dtype=jnp.int32) // TOPK
    onehot = (eid[:, None] == jnp.arange(E, dtype=jnp.int32)[None, :])
    onehot = onehot.astype(jnp.int32)
    ranks = jnp.cumsum(onehot, axis=0) - onehot
    rank = jnp.take_along_axis(ranks, eid[:, None], axis=1)[:, 0]
    counts = jnp.sum(onehot, axis=0)
    padded = ((counts + BT - 1) // BT) * BT
    astart = jnp.concatenate(
        [jnp.zeros((1,), jnp.int32), jnp.cumsum(padded)[:-1]])
    pos = astart[eid] + rank
    tok_src = jnp.zeros((NB * BT,), jnp.int32).at[pos].set(tok)
    ws_arr = jnp.zeros((NB * BT,), jnp.float32).at[pos].set(wslot)
    ws_arr = ws_arr.reshape(NB * BT, 1)
    bstart = jnp.arange(NB, dtype=jnp.int32) * BT
    gend = astart + padded
    beid = jnp.minimum(
        jnp.sum((bstart[:, None] >= gend[None, :]).astype(jnp.int32), axis=1),
        E - 1)
    bval = (bstart < (astart + counts)[beid]).astype(jnp.int32)

    xs = _dispatch(x, tok_src)
    hs = _ffn1(beid, bval, xs, ws_arr, w1.astype(jnp.bfloat16),
               w3.astype(jnp.bfloat16))
    y = _ffn2(beid, bval, hs, w2.astype(jnp.bfloat16))
    pos2 = pos.reshape(T, TOPK)
    out = _combine(y, pos2[:, 0], pos2[:, 1])
    return (out.reshape(orig_shape), logits)


# speedup vs baseline: 1.3887x; 1.3887x over previous
import functools

import jax
import jax.numpy as jnp
from jax import lax
from jax.experimental import pallas as pl
from jax.experimental.pallas import tpu as pltpu
from jax.experimental.pallas import tpu_sc as plsc

H = 1024
F = 3584
E = 8
TOPK = 2
T = 2048
S = T * TOPK
BT = 256
NB = S // BT + E
NBT = NB * BT
FT = 512
NF = F // FT

NC = 2
NS = 16
NW = NC * NS

_SC_MESH = plsc.VectorSubcoreMesh(core_axis_name="c", subcore_axis_name="s")


TB = 256


def _router_body(x_ref, gw_ref, logits_ref, idx_ref, w_ref):
    x = x_ref[...]
    gw = gw_ref[...]
    logits = lax.dot_general(x, gw, (((1,), (1,)), ((), ())),
                             preferred_element_type=jnp.float32)
    logits_ref[...] = logits
    lane = lax.broadcasted_iota(jnp.int32, (TB, E), 1)
    m1 = jnp.max(logits, axis=1, keepdims=True)
    i1 = jnp.min(jnp.where(logits == m1, lane, E), axis=1, keepdims=True)
    masked = jnp.where(lane == i1, -jnp.float32(1e30), logits)
    m2 = jnp.max(masked, axis=1, keepdims=True)
    i2 = jnp.min(jnp.where(masked == m2, lane, E), axis=1, keepdims=True)
    p1 = 1.0 / (1.0 + jnp.exp(m2 - m1))
    idx_ref[...] = jnp.concatenate([i1, i2], axis=1)
    w_ref[...] = jnp.concatenate([p1, 1.0 - p1], axis=1)


def _router(x, gate_w):
    return pl.pallas_call(
        _router_body,
        grid=(T // TB,),
        in_specs=[
            pl.BlockSpec((TB, H), lambda i: (i, 0)),
            pl.BlockSpec((E, H), lambda i: (0, 0)),
        ],
        out_specs=[
            pl.BlockSpec((TB, E), lambda i: (i, 0)),
            pl.BlockSpec((TB, TOPK), lambda i: (i, 0)),
            pl.BlockSpec((TB, TOPK), lambda i: (i, 0)),
        ],
        out_shape=[
            jax.ShapeDtypeStruct((T, E), jnp.float32),
            jax.ShapeDtypeStruct((T, TOPK), jnp.int32),
            jax.ShapeDtypeStruct((T, TOPK), jnp.float32),
        ],
    )(x, gate_w)


RPW = NBT // NW
DCH = 48
DNCH = RPW // DCH


@functools.partial(
    pl.kernel,
    mesh=_SC_MESH,
    out_type=jax.ShapeDtypeStruct((NBT, H), jnp.float32),
    scratch_types=[
        pltpu.VMEM((DCH,), jnp.int32),
        pltpu.VMEM((DCH,), jnp.int32),
        pltpu.VMEM((DCH, H), jnp.float32),
        pltpu.VMEM((DCH, H), jnp.float32),
        pltpu.SemaphoreType.DMA,
        pltpu.SemaphoreType.DMA,
    ],
)
def _dispatch(x_hbm, idx_hbm, xs_hbm, i0, i1, b0, b1, s0, s1):
    wid = lax.axis_index("s") * NC + lax.axis_index("c")
    base = wid * RPW
    ibufs = (i0, i1)
    bufs = (b0, b1)
    sems = (s0, s1)
    cps = []
    for k in range(2):
        pltpu.sync_copy(idx_hbm.at[pl.ds(base + k * DCH, DCH)], ibufs[k])
        cps.append(pltpu.async_copy(x_hbm.at[ibufs[k]], bufs[k], sems[k]))
    for c in range(DNCH):
        k = c % 2
        cps[k].wait()
        pltpu.sync_copy(bufs[k], xs_hbm.at[pl.ds(base + c * DCH, DCH)])
        nxt = c + 2
        if nxt < DNCH:
            pltpu.sync_copy(idx_hbm.at[pl.ds(base + nxt * DCH, DCH)],
                            ibufs[k])
            cps[k] = pltpu.async_copy(x_hbm.at[ibufs[k]], bufs[k], sems[k])


def _ffn1_body(seid_ref, sval_ref, xs_ref, ws_ref, w1_ref, w3_ref, hs_ref,
               w1bf, w3bf):
    b = pl.program_id(1)
    is_new = jnp.logical_or(
        b == 0, seid_ref[b] != seid_ref[jnp.maximum(b - 1, 0)])

    @pl.when(is_new)
    def _():
        w1bf[...] = w1_ref[0].astype(jnp.bfloat16)
        w3bf[...] = w3_ref[0].astype(jnp.bfloat16)

    @pl.when(sval_ref[b] > 0)
    def _():
        x = xs_ref[...].astype(jnp.bfloat16)
        g = lax.dot_general(x, w1bf[...], (((1,), (1,)), ((), ())),
                            preferred_element_type=jnp.float32)
        u = lax.dot_general(x, w3bf[...], (((1,), (1,)), ((), ())),
                            preferred_element_type=jnp.float32)
        h = (g * jax.nn.sigmoid(g)) * (u * ws_ref[...])
        hs_ref[...] = h.astype(jnp.bfloat16)


def _ffn1(beid, bval, xs, ws, w1, w3):
    grid_spec = pltpu.PrefetchScalarGridSpec(
        num_scalar_prefetch=2,
        grid=(NF, NB),
        in_specs=[
            pl.BlockSpec((BT, H), lambda f, b, seid, sval: (b, 0)),
            pl.BlockSpec((BT, 1), lambda f, b, seid, sval: (b, 0)),
            pl.BlockSpec((1, FT, H), lambda f, b, seid, sval: (seid[b], f, 0)),
            pl.BlockSpec((1, FT, H), lambda f, b, seid, sval: (seid[b], f, 0)),
        ],
        out_specs=pl.BlockSpec((BT, FT), lambda f, b, seid, sval: (b, f)),
        scratch_shapes=[
            pltpu.VMEM((FT, H), jnp.bfloat16),
            pltpu.VMEM((FT, H), jnp.bfloat16),
        ],
    )
    return pl.pallas_call(
        _ffn1_body,
        grid_spec=grid_spec,
        out_shape=jax.ShapeDtypeStruct((NBT, F), jnp.bfloat16),
        compiler_params=pltpu.CompilerParams(
            dimension_semantics=("arbitrary", "arbitrary")),
    )(beid, bval, xs, ws, w1, w3)


def _ffn2_body(seid_ref, sval_ref, hs_ref, w2_ref, y_ref, w2bf):
    b = pl.program_id(0)
    is_new = jnp.logical_or(
        b == 0, seid_ref[b] != seid_ref[jnp.maximum(b - 1, 0)])

    @pl.when(is_new)
    def _():
        w2bf[...] = w2_ref[0].astype(jnp.bfloat16)

    @pl.when(sval_ref[b] > 0)
    def _():
        y_ref[...] = lax.dot_general(
            hs_ref[...], w2bf[...], (((1,), (1,)), ((), ())),
            preferred_element_type=jnp.float32)


def _ffn2(beid, bval, hs, w2):
    grid_spec = pltpu.PrefetchScalarGridSpec(
        num_scalar_prefetch=2,
        grid=(NB,),
        in_specs=[
            pl.BlockSpec((BT, F), lambda b, seid, sval: (b, 0)),
            pl.BlockSpec((1, H, F), lambda b, seid, sval: (seid[b], 0, 0)),
        ],
        out_specs=pl.BlockSpec((BT, H), lambda b, seid, sval: (b, 0)),
        scratch_shapes=[
            pltpu.VMEM((H, F), jnp.bfloat16),
        ],
    )
    return pl.pallas_call(
        _ffn2_body,
        grid_spec=grid_spec,
        out_shape=jax.ShapeDtypeStruct((NBT, H), jnp.float32),
        compiler_params=pltpu.CompilerParams(
            dimension_semantics=("arbitrary",)),
    )(beid, bval, hs, w2)


TPW = T // NW
CCH = 16
CNCH = TPW // CCH


@functools.partial(
    pl.kernel,
    mesh=_SC_MESH,
    out_type=jax.ShapeDtypeStruct((T, H), jnp.float32),
    scratch_types=[
        pltpu.VMEM((CCH,), jnp.int32),
        pltpu.VMEM((CCH,), jnp.int32),
        pltpu.VMEM((CCH,), jnp.int32),
        pltpu.VMEM((CCH,), jnp.int32),
        pltpu.VMEM((CCH, H), jnp.float32),
        pltpu.VMEM((CCH, H), jnp.float32),
        pltpu.VMEM((CCH, H), jnp.float32),
        pltpu.VMEM((CCH, H), jnp.float32),
        pltpu.SemaphoreType.DMA,
        pltpu.SemaphoreType.DMA,
        pltpu.SemaphoreType.DMA,
        pltpu.SemaphoreType.DMA,
    ],
)
def _combine(y_hbm, pos0_hbm, pos1_hbm, out_hbm, ia0, ib0, ia1, ib1,
             a0, b0, a1, b1, sa0, sb0, sa1, sb1):
    wid = lax.axis_index("s") * NC + lax.axis_index("c")
    base = wid * TPW
    iabufs = (ia0, ia1)
    ibbufs = (ib0, ib1)
    abufs = (a0, a1)
    bbufs = (b0, b1)
    sas = (sa0, sa1)
    sbs = (sb0, sb1)

    def start(c, k):
        off = base + c * CCH
        pltpu.sync_copy(pos0_hbm.at[pl.ds(off, CCH)], iabufs[k])
        pltpu.sync_copy(pos1_hbm.at[pl.ds(off, CCH)], ibbufs[k])
        return (pltpu.async_copy(y_hbm.at[iabufs[k]], abufs[k], sas[k]),
                pltpu.async_copy(y_hbm.at[ibbufs[k]], bbufs[k], sbs[k]))

    cps = [start(0, 0), start(1, 1)]
    for c in range(CNCH):
        k = c % 2
        cps[k][0].wait()
        cps[k][1].wait()
        av, bv = abufs[k], bbufs[k]

        def row(r, rc):
            for j in range(H // 16):
                sl = pl.ds(j * 16, 16)
                av[r, sl] = av[r, sl] + bv[r, sl]
            return rc

        lax.fori_loop(0, CCH, row, 0)
        pltpu.sync_copy(av, out_hbm.at[pl.ds(base + c * CCH, CCH)])
        if c + 2 < CNCH:
            cps[k] = start(c + 2, k)


def kernel(hidden_states, gate_w, w1, w3, w2):
    orig_shape = hidden_states.shape
    x = hidden_states.reshape(T, H)

    logits, top_idx, top_w = _router(x, gate_w)

    eid = top_idx.reshape(S)
    wslot = top_w.reshape(S)
    tok = jnp.arange(S, dtype=jnp.int32) // TOPK
    onehot = (eid[:, None] == jnp.arange(E, dtype=jnp.int32)[None, :])
    onehot = onehot.astype(jnp.int32)
    ranks = jnp.cumsum(onehot, axis=0) - onehot
    rank = jnp.take_along_axis(ranks, eid[:, None], axis=1)[:, 0]
    counts = jnp.sum(onehot, axis=0)
    padded = ((counts + BT - 1) // BT) * BT
    astart = jnp.concatenate(
        [jnp.zeros((1,), jnp.int32), jnp.cumsum(padded)[:-1]])
    pos = astart[eid] + rank
    tok_src = jnp.zeros((NBT,), jnp.int32).at[pos].set(tok)
    ws_arr = jnp.zeros((NBT,), jnp.float32).at[pos].set(wslot)
    ws_arr = ws_arr.reshape(NBT, 1)
    bstart = jnp.arange(NB, dtype=jnp.int32) * BT
    gend = astart + padded
    beid = jnp.minimum(
        jnp.sum((bstart[:, None] >= gend[None, :]).astype(jnp.int32), axis=1),
        E - 1)
    bval = (bstart < (astart + counts)[beid]).astype(jnp.int32)

    xs = _dispatch(x, tok_src)
    hs = _ffn1(beid, bval, xs, ws_arr, w1, w3)
    y = _ffn2(beid, bval, hs, w2)
    pos2 = pos.reshape(T, TOPK)
    out = _combine(y, pos2[:, 0], pos2[:, 1])
    return (out.reshape(orig_shape), logits)


# speedup vs baseline: 1.6748x; 1.2060x over previous
import functools

import jax
import jax.numpy as jnp
from jax import lax
from jax.experimental import pallas as pl
from jax.experimental.pallas import tpu as pltpu
from jax.experimental.pallas import tpu_sc as plsc

H = 1024
F = 3584
E = 8
TOPK = 2
T = 2048
S = T * TOPK
BT = 256
NB = S // BT + E
NBT = NB * BT
FT = 512
NF = F // FT

NC = 2
NS = 16
NW = NC * NS

_SC_MESH = plsc.VectorSubcoreMesh(core_axis_name="c", subcore_axis_name="s")


TB = 256


def _router_body(x_ref, gw_ref, logits_ref, idx_ref, w0_ref, w1_ref):
    x = x_ref[...]
    gw = gw_ref[...]
    logits = lax.dot_general(x, gw, (((1,), (1,)), ((), ())),
                             preferred_element_type=jnp.float32)
    logits_ref[...] = logits
    lane = lax.broadcasted_iota(jnp.int32, (TB, E), 1)
    m1 = jnp.max(logits, axis=1, keepdims=True)
    i1 = jnp.min(jnp.where(logits == m1, lane, E), axis=1, keepdims=True)
    masked = jnp.where(lane == i1, -jnp.float32(1e30), logits)
    m2 = jnp.max(masked, axis=1, keepdims=True)
    i2 = jnp.min(jnp.where(masked == m2, lane, E), axis=1, keepdims=True)
    p1 = 1.0 / (1.0 + jnp.exp(m2 - m1))
    idx_ref[...] = jnp.concatenate([i1, i2], axis=1)
    w0_ref[...] = jnp.broadcast_to(p1, (TB, 16))
    w1_ref[...] = jnp.broadcast_to(1.0 - p1, (TB, 16))


def _router(x, gate_w):
    return pl.pallas_call(
        _router_body,
        grid=(T // TB,),
        in_specs=[
            pl.BlockSpec((TB, H), lambda i: (i, 0)),
            pl.BlockSpec((E, H), lambda i: (0, 0)),
        ],
        out_specs=[
            pl.BlockSpec((TB, E), lambda i: (i, 0)),
            pl.BlockSpec((TB, TOPK), lambda i: (i, 0)),
            pl.BlockSpec((TB, 16), lambda i: (i, 0)),
            pl.BlockSpec((TB, 16), lambda i: (i, 0)),
        ],
        out_shape=[
            jax.ShapeDtypeStruct((T, E), jnp.float32),
            jax.ShapeDtypeStruct((T, TOPK), jnp.int32),
            jax.ShapeDtypeStruct((T, 16), jnp.float32),
            jax.ShapeDtypeStruct((T, 16), jnp.float32),
        ],
    )(x, gate_w)


TPW = T // NW
DCH = 32
DNCH = TPW // DCH


@functools.partial(
    pl.kernel,
    mesh=_SC_MESH,
    out_type=jax.ShapeDtypeStruct((NBT, H), jnp.float32),
    scratch_types=[
        pltpu.VMEM((DCH,), jnp.int32),
        pltpu.VMEM((DCH,), jnp.int32),
        pltpu.VMEM((DCH,), jnp.int32),
        pltpu.VMEM((DCH,), jnp.int32),
        pltpu.VMEM((DCH, H), jnp.float32),
        pltpu.VMEM((DCH, H), jnp.float32),
        pltpu.SemaphoreType.DMA,
        pltpu.SemaphoreType.DMA,
        pltpu.SemaphoreType.DMA,
        pltpu.SemaphoreType.DMA,
        pltpu.SemaphoreType.DMA,
        pltpu.SemaphoreType.DMA,
    ],
)
def _dispatch(x_hbm, pos0_hbm, pos1_hbm, xs_hbm, p0a, p1a, p0b, p1b,
              bufa, bufb, sra, srb, sw0a, sw1a, sw0b, sw1b):
    wid = lax.axis_index("s") * NC + lax.axis_index("c")
    base = wid * TPW
    p0s = (p0a, p0b)
    p1s = (p1a, p1b)
    bufs = (bufa, bufb)
    srs = (sra, srb)
    sw0s = (sw0a, sw0b)
    sw1s = (sw1a, sw1b)
    reads = []
    for k in range(DNCH):
        off = base + k * DCH
        pltpu.sync_copy(pos0_hbm.at[pl.ds(off, DCH)], p0s[k])
        pltpu.sync_copy(pos1_hbm.at[pl.ds(off, DCH)], p1s[k])
        reads.append(
            pltpu.async_copy(x_hbm.at[pl.ds(off, DCH)], bufs[k], srs[k]))
    writes = []
    for k in range(DNCH):
        reads[k].wait()
        writes.append(
            pltpu.async_copy(bufs[k], xs_hbm.at[p0s[k]], sw0s[k]))
        writes.append(
            pltpu.async_copy(bufs[k], xs_hbm.at[p1s[k]], sw1s[k]))
    for w in writes:
        w.wait()


def _ffn_body(seid_ref, sval_ref, xs_ref, w1_ref, w3_ref, w2_ref, y_ref,
              w1bf, w3bf, w2bf, yacc):
    f = pl.program_id(0)
    b = pl.program_id(1)
    is_new = jnp.logical_or(
        b == 0, seid_ref[b] != seid_ref[jnp.maximum(b - 1, 0)])

    @pl.when(is_new)
    def _():
        w1bf[...] = w1_ref[0].astype(jnp.bfloat16)
        w3bf[...] = w3_ref[0].astype(jnp.bfloat16)
        w2bf[...] = w2_ref[0].astype(jnp.bfloat16)

    @pl.when(sval_ref[b] > 0)
    def _():
        x = xs_ref[...].astype(jnp.bfloat16)
        g = lax.dot_general(x, w1bf[...], (((1,), (1,)), ((), ())),
                            preferred_element_type=jnp.float32)
        u = lax.dot_general(x, w3bf[...], (((1,), (1,)), ((), ())),
                            preferred_element_type=jnp.float32)
        h = ((g * jax.nn.sigmoid(g)) * u).astype(jnp.bfloat16)
        part = lax.dot_general(h, w2bf[...], (((1,), (1,)), ((), ())),
                               preferred_element_type=jnp.float32)
        sl = pl.ds(b * BT, BT)

        @pl.when(f == 0)
        def _():
            yacc[sl, :] = part

        @pl.when(jnp.logical_and(f > 0, f < NF - 1))
        def _():
            yacc[sl, :] += part

        @pl.when(f == NF - 1)
        def _():
            y_ref[...] = yacc[sl, :] + part


def _ffn(beid, bval, xs, w1, w3, w2):
    grid_spec = pltpu.PrefetchScalarGridSpec(
        num_scalar_prefetch=2,
        grid=(NF, NB),
        in_specs=[
            pl.BlockSpec((BT, H), lambda f, b, seid, sval: (b, 0)),
            pl.BlockSpec((1, FT, H), lambda f, b, seid, sval: (seid[b], f, 0)),
            pl.BlockSpec((1, FT, H), lambda f, b, seid, sval: (seid[b], f, 0)),
            pl.BlockSpec((1, H, FT), lambda f, b, seid, sval: (seid[b], 0, f)),
        ],
        out_specs=pl.BlockSpec((BT, H), lambda f, b, seid, sval: (b, 0)),
        scratch_shapes=[
            pltpu.VMEM((FT, H), jnp.bfloat16),
            pltpu.VMEM((FT, H), jnp.bfloat16),
            pltpu.VMEM((H, FT), jnp.bfloat16),
            pltpu.VMEM((NBT, H), jnp.float32),
        ],
    )
    return pl.pallas_call(
        _ffn_body,
        grid_spec=grid_spec,
        out_shape=jax.ShapeDtypeStruct((NBT, H), jnp.float32),
        compiler_params=pltpu.CompilerParams(
            dimension_semantics=("arbitrary", "arbitrary")),
    )(beid, bval, xs, w1, w3, w2)


CCH = 16
CNCH = TPW // CCH


@functools.partial(
    pl.kernel,
    mesh=_SC_MESH,
    out_type=jax.ShapeDtypeStruct((T, H), jnp.float32),
    scratch_types=[
        pltpu.VMEM((CCH,), jnp.int32),
        pltpu.VMEM((CCH,), jnp.int32),
        pltpu.VMEM((CCH,), jnp.int32),
        pltpu.VMEM((CCH,), jnp.int32),
        pltpu.VMEM((CCH, 16), jnp.float32),
        pltpu.VMEM((CCH, 16), jnp.float32),
        pltpu.VMEM((CCH, 16), jnp.float32),
        pltpu.VMEM((CCH, 16), jnp.float32),
        pltpu.VMEM((CCH, H), jnp.float32),
        pltpu.VMEM((CCH, H), jnp.float32),
        pltpu.VMEM((CCH, H), jnp.float32),
        pltpu.VMEM((CCH, H), jnp.float32),
        pltpu.SemaphoreType.DMA,
        pltpu.SemaphoreType.DMA,
        pltpu.SemaphoreType.DMA,
        pltpu.SemaphoreType.DMA,
    ],
)
def _combine(y_hbm, pos0_hbm, pos1_hbm, wt0_hbm, wt1_hbm, out_hbm,
             ia0, ib0, ia1, ib1, wa0, wb0, wa1, wb1,
             a0, b0, a1, b1, sa0, sb0, sa1, sb1):
    wid = lax.axis_index("s") * NC + lax.axis_index("c")
    base = wid * TPW
    iabufs = (ia0, ia1)
    ibbufs = (ib0, ib1)
    wabufs = (wa0, wa1)
    wbbufs = (wb0, wb1)
    abufs = (a0, a1)
    bbufs = (b0, b1)
    sas = (sa0, sa1)
    sbs = (sb0, sb1)

    def start(c, k):
        off = base + c * CCH
        pltpu.sync_copy(pos0_hbm.at[pl.ds(off, CCH)], iabufs[k])
        pltpu.sync_copy(pos1_hbm.at[pl.ds(off, CCH)], ibbufs[k])
        pltpu.sync_copy(wt0_hbm.at[pl.ds(off, CCH)], wabufs[k])
        pltpu.sync_copy(wt1_hbm.at[pl.ds(off, CCH)], wbbufs[k])
        return (pltpu.async_copy(y_hbm.at[iabufs[k]], abufs[k], sas[k]),
                pltpu.async_copy(y_hbm.at[ibbufs[k]], bbufs[k], sbs[k]))

    cps = [start(0, 0), start(1, 1)]
    for c in range(CNCH):
        k = c % 2
        cps[k][0].wait()
        cps[k][1].wait()
        av, bv = abufs[k], bbufs[k]
        wav, wbv = wabufs[k], wbbufs[k]

        def row(r, rc):
            w0r = wav[r, :]
            w1r = wbv[r, :]
            for j in range(H // 16):
                sl = pl.ds(j * 16, 16)
                av[r, sl] = av[r, sl] * w0r + bv[r, sl] * w1r
            return rc

        lax.fori_loop(0, CCH, row, 0)
        pltpu.sync_copy(av, out_hbm.at[pl.ds(base + c * CCH, CCH)])
        if c + 2 < CNCH:
            cps[k] = start(c + 2, k)


def kernel(hidden_states, gate_w, w1, w3, w2):
    orig_shape = hidden_states.shape
    x = hidden_states.reshape(T, H)

    logits, top_idx, w16a, w16b = _router(x, gate_w)

    eid = top_idx.reshape(S)
    onehot = (eid[:, None] == jnp.arange(E, dtype=jnp.int32)[None, :])
    onehot = onehot.astype(jnp.int32)
    ranks = jnp.cumsum(onehot, axis=0) - onehot
    rank = jnp.take_along_axis(ranks, eid[:, None], axis=1)[:, 0]
    counts = jnp.sum(onehot, axis=0)
    padded = ((counts + BT - 1) // BT) * BT
    astart = jnp.concatenate(
        [jnp.zeros((1,), jnp.int32), jnp.cumsum(padded)[:-1]])
    pos = astart[eid] + rank
    bstart = jnp.arange(NB, dtype=jnp.int32) * BT
    gend = astart + padded
    beid = jnp.minimum(
        jnp.sum((bstart[:, None] >= gend[None, :]).astype(jnp.int32), axis=1),
        E - 1)
    bval = (bstart < (astart + counts)[beid]).astype(jnp.int32)
    pos2 = pos.reshape(T, TOPK)
    pos0 = pos2[:, 0]
    pos1 = pos2[:, 1]

    xs = _dispatch(x, pos0, pos1)
    y = _ffn(beid, bval, xs, w1, w3, w2)
    out = _combine(y, pos0, pos1, w16a, w16b)
    return (out.reshape(orig_shape), logits)


# speedup vs baseline: 1.7933x; 1.0708x over previous
import functools

import jax
import jax.numpy as jnp
from jax import lax
from jax.experimental import pallas as pl
from jax.experimental.pallas import tpu as pltpu
from jax.experimental.pallas import tpu_sc as plsc

H = 1024
F = 3584
E = 8
TOPK = 2
T = 2048
S = T * TOPK
BT = 256
NB = S // BT + E
NBT = NB * BT
FT = 512
NF = F // FT

NC = 2
NS = 16
NW = NC * NS

_SC_MESH = plsc.VectorSubcoreMesh(core_axis_name="c", subcore_axis_name="s")


TB = 256


def _router_body(x_ref, gw_ref, logits_ref, idx_ref, w0_ref, w1_ref):
    x = x_ref[...]
    gw = gw_ref[...]
    logits = lax.dot_general(x, gw, (((1,), (1,)), ((), ())),
                             preferred_element_type=jnp.float32)
    logits_ref[...] = logits
    lane = lax.broadcasted_iota(jnp.int32, (TB, E), 1)
    m1 = jnp.max(logits, axis=1, keepdims=True)
    i1 = jnp.min(jnp.where(logits == m1, lane, E), axis=1, keepdims=True)
    masked = jnp.where(lane == i1, -jnp.float32(1e30), logits)
    m2 = jnp.max(masked, axis=1, keepdims=True)
    i2 = jnp.min(jnp.where(masked == m2, lane, E), axis=1, keepdims=True)
    p1 = 1.0 / (1.0 + jnp.exp(m2 - m1))
    idx_ref[...] = jnp.concatenate([i1, i2], axis=1)
    w0_ref[...] = jnp.broadcast_to(p1, (TB, 16))
    w1_ref[...] = jnp.broadcast_to(1.0 - p1, (TB, 16))


def _router(x, gate_w):
    return pl.pallas_call(
        _router_body,
        grid=(T // TB,),
        in_specs=[
            pl.BlockSpec((TB, H), lambda i: (i, 0)),
            pl.BlockSpec((E, H), lambda i: (0, 0)),
        ],
        out_specs=[
            pl.BlockSpec((TB, E), lambda i: (i, 0)),
            pl.BlockSpec((TB, TOPK), lambda i: (i, 0)),
            pl.BlockSpec((TB, 16), lambda i: (i, 0)),
            pl.BlockSpec((TB, 16), lambda i: (i, 0)),
        ],
        out_shape=[
            jax.ShapeDtypeStruct((T, E), jnp.float32),
            jax.ShapeDtypeStruct((T, TOPK), jnp.int32),
            jax.ShapeDtypeStruct((T, 16), jnp.float32),
            jax.ShapeDtypeStruct((T, 16), jnp.float32),
        ],
    )(x, gate_w)


TPW = T // NW
DCH = 32
DNCH = TPW // DCH


@functools.partial(
    pl.kernel,
    mesh=_SC_MESH,
    out_type=jax.ShapeDtypeStruct((NBT, H), jnp.float32),
    scratch_types=[
        pltpu.VMEM((DCH,), jnp.int32),
        pltpu.VMEM((DCH,), jnp.int32),
        pltpu.VMEM((DCH,), jnp.int32),
        pltpu.VMEM((DCH,), jnp.int32),
        pltpu.VMEM((DCH, H), jnp.float32),
        pltpu.VMEM((DCH, H), jnp.float32),
        pltpu.SemaphoreType.DMA,
        pltpu.SemaphoreType.DMA,
        pltpu.SemaphoreType.DMA,
        pltpu.SemaphoreType.DMA,
        pltpu.SemaphoreType.DMA,
        pltpu.SemaphoreType.DMA,
    ],
)
def _dispatch(x_hbm, pos0_hbm, pos1_hbm, xs_hbm, p0a, p1a, p0b, p1b,
              bufa, bufb, sra, srb, sw0a, sw1a, sw0b, sw1b):
    wid = lax.axis_index("s") * NC + lax.axis_index("c")
    base = wid * TPW
    p0s = (p0a, p0b)
    p1s = (p1a, p1b)
    bufs = (bufa, bufb)
    srs = (sra, srb)
    sw0s = (sw0a, sw0b)
    sw1s = (sw1a, sw1b)
    reads = []
    for k in range(DNCH):
        off = base + k * DCH
        pltpu.sync_copy(pos0_hbm.at[pl.ds(off, DCH)], p0s[k])
        pltpu.sync_copy(pos1_hbm.at[pl.ds(off, DCH)], p1s[k])
        reads.append(
            pltpu.async_copy(x_hbm.at[pl.ds(off, DCH)], bufs[k], srs[k]))
    writes = []
    for k in range(DNCH):
        reads[k].wait()
        writes.append(
            pltpu.async_copy(bufs[k], xs_hbm.at[p0s[k]], sw0s[k]))
        writes.append(
            pltpu.async_copy(bufs[k], xs_hbm.at[p1s[k]], sw1s[k]))
    for w in writes:
        w.wait()


def _cast_body(xs_ref, xb_ref):
    xb_ref[...] = xs_ref[...].astype(jnp.bfloat16)


def _cast_bf16(xs):
    return pl.pallas_call(
        _cast_body,
        grid=(NB,),
        in_specs=[pl.BlockSpec((BT, H), lambda b: (b, 0))],
        out_specs=pl.BlockSpec((BT, H), lambda b: (b, 0)),
        out_shape=jax.ShapeDtypeStruct((NBT, H), jnp.bfloat16),
    )(xs)


def _ffn_body(seid_ref, sval_ref, xs_ref, w1_ref, w3_ref, w2_ref, y_ref,
              w1bf, w3bf, w2bf, yacc):
    f = pl.program_id(0)
    b = pl.program_id(1)
    is_new = jnp.logical_or(
        b == 0, seid_ref[b] != seid_ref[jnp.maximum(b - 1, 0)])

    @pl.when(is_new)
    def _():
        w1bf[...] = w1_ref[0].astype(jnp.bfloat16)
        w3bf[...] = w3_ref[0].astype(jnp.bfloat16)
        w2bf[...] = w2_ref[0].astype(jnp.bfloat16)

    @pl.when(sval_ref[b] > 0)
    def _():
        x = xs_ref[...]
        g = lax.dot_general(x, w1bf[...], (((1,), (1,)), ((), ())),
                            preferred_element_type=jnp.float32)
        u = lax.dot_general(x, w3bf[...], (((1,), (1,)), ((), ())),
                            preferred_element_type=jnp.float32)
        h = ((g * jax.nn.sigmoid(g)) * u).astype(jnp.bfloat16)
        part = lax.dot_general(h, w2bf[...], (((1,), (1,)), ((), ())),
                               preferred_element_type=jnp.float32)
        sl = pl.ds(b * BT, BT)

        @pl.when(f == 0)
        def _():
            yacc[sl, :] = part

        @pl.when(jnp.logical_and(f > 0, f < NF - 1))
        def _():
            yacc[sl, :] += part

        @pl.when(f == NF - 1)
        def _():
            y_ref[...] = yacc[sl, :] + part


def _ffn(beid, bval, xs, w1, w3, w2):
    grid_spec = pltpu.PrefetchScalarGridSpec(
        num_scalar_prefetch=2,
        grid=(NF, NB),
        in_specs=[
            pl.BlockSpec((BT, H), lambda f, b, seid, sval: (b, 0)),
            pl.BlockSpec((1, FT, H), lambda f, b, seid, sval: (seid[b], f, 0)),
            pl.BlockSpec((1, FT, H), lambda f, b, seid, sval: (seid[b], f, 0)),
            pl.BlockSpec((1, H, FT), lambda f, b, seid, sval: (seid[b], 0, f)),
        ],
        out_specs=pl.BlockSpec(
            (BT, H),
            lambda f, b, seid, sval: (jnp.where(f == NF - 1, b, 0), 0)),
        scratch_shapes=[
            pltpu.VMEM((FT, H), jnp.bfloat16),
            pltpu.VMEM((FT, H), jnp.bfloat16),
            pltpu.VMEM((H, FT), jnp.bfloat16),
            pltpu.VMEM((NBT, H), jnp.float32),
        ],
    )
    return pl.pallas_call(
        _ffn_body,
        grid_spec=grid_spec,
        out_shape=jax.ShapeDtypeStruct((NBT, H), jnp.float32),
        compiler_params=pltpu.CompilerParams(
            dimension_semantics=("arbitrary", "arbitrary")),
    )(beid, bval, xs, w1, w3, w2)


CCH = 16
CNCH = TPW // CCH


@functools.partial(
    pl.kernel,
    mesh=_SC_MESH,
    out_type=jax.ShapeDtypeStruct((T, H), jnp.float32),
    scratch_types=[
        pltpu.VMEM((CCH,), jnp.int32),
        pltpu.VMEM((CCH,), jnp.int32),
        pltpu.VMEM((CCH,), jnp.int32),
        pltpu.VMEM((CCH,), jnp.int32),
        pltpu.VMEM((CCH, 16), jnp.float32),
        pltpu.VMEM((CCH, 16), jnp.float32),
        pltpu.VMEM((CCH, 16), jnp.float32),
        pltpu.VMEM((CCH, 16), jnp.float32),
        pltpu.VMEM((CCH, H), jnp.float32),
        pltpu.VMEM((CCH, H), jnp.float32),
        pltpu.VMEM((CCH, H), jnp.float32),
        pltpu.VMEM((CCH, H), jnp.float32),
        pltpu.SemaphoreType.DMA,
        pltpu.SemaphoreType.DMA,
        pltpu.SemaphoreType.DMA,
        pltpu.SemaphoreType.DMA,
    ],
)
def _combine(y_hbm, pos0_hbm, pos1_hbm, wt0_hbm, wt1_hbm, out_hbm,
             ia0, ib0, ia1, ib1, wa0, wb0, wa1, wb1,
             a0, b0, a1, b1, sa0, sb0, sa1, sb1):
    wid = lax.axis_index("s") * NC + lax.axis_index("c")
    base = wid * TPW
    iabufs = (ia0, ia1)
    ibbufs = (ib0, ib1)
    wabufs = (wa0, wa1)
    wbbufs = (wb0, wb1)
    abufs = (a0, a1)
    bbufs = (b0, b1)
    sas = (sa0, sa1)
    sbs = (sb0, sb1)

    def start(c, k):
        off = base + c * CCH
        pltpu.sync_copy(pos0_hbm.at[pl.ds(off, CCH)], iabufs[k])
        pltpu.sync_copy(pos1_hbm.at[pl.ds(off, CCH)], ibbufs[k])
        pltpu.sync_copy(wt0_hbm.at[pl.ds(off, CCH)], wabufs[k])
        pltpu.sync_copy(wt1_hbm.at[pl.ds(off, CCH)], wbbufs[k])
        return (pltpu.async_copy(y_hbm.at[iabufs[k]], abufs[k], sas[k]),
                pltpu.async_copy(y_hbm.at[ibbufs[k]], bbufs[k], sbs[k]))

    cps = [start(0, 0), start(1, 1)]
    for c in range(CNCH):
        k = c % 2
        cps[k][0].wait()
        cps[k][1].wait()
        av, bv = abufs[k], bbufs[k]
        wav, wbv = wabufs[k], wbbufs[k]

        def row(r, rc):
            w0r = wav[r, :]
            w1r = wbv[r, :]
            for j in range(H // 16):
                sl = pl.ds(j * 16, 16)
                av[r, sl] = av[r, sl] * w0r + bv[r, sl] * w1r
            return rc

        lax.fori_loop(0, CCH, row, 0)
        pltpu.sync_copy(av, out_hbm.at[pl.ds(base + c * CCH, CCH)])
        if c + 2 < CNCH:
            cps[k] = start(c + 2, k)


def kernel(hidden_states, gate_w, w1, w3, w2):
    orig_shape = hidden_states.shape
    x = hidden_states.reshape(T, H)

    logits, top_idx, w16a, w16b = _router(x, gate_w)

    eid = top_idx.reshape(S)
    onehot = (eid[:, None] == jnp.arange(E, dtype=jnp.int32)[None, :])
    onehot = onehot.astype(jnp.int32)
    ranks = jnp.cumsum(onehot, axis=0) - onehot
    rank = jnp.take_along_axis(ranks, eid[:, None], axis=1)[:, 0]
    counts = jnp.sum(onehot, axis=0)
    padded = ((counts + BT - 1) // BT) * BT
    astart = jnp.concatenate(
        [jnp.zeros((1,), jnp.int32), jnp.cumsum(padded)[:-1]])
    pos = astart[eid] + rank
    bstart = jnp.arange(NB, dtype=jnp.int32) * BT
    gend = astart + padded
    beid = jnp.minimum(
        jnp.sum((bstart[:, None] >= gend[None, :]).astype(jnp.int32), axis=1),
        E - 1)
    bval = (bstart < (astart + counts)[beid]).astype(jnp.int32)
    pos2 = pos.reshape(T, TOPK)
    pos0 = pos2[:, 0]
    pos1 = pos2[:, 1]

    xs = _dispatch(x, pos0, pos1)
    y = _ffn(beid, bval, _cast_bf16(xs), w1, w3, w2)
    out = _combine(y, pos0, pos1, w16a, w16b)
    return (out.reshape(orig_shape), logits)


# speedup vs baseline: 1.9766x; 1.1022x over previous
import functools

import jax
import jax.numpy as jnp
from jax import lax
from jax.experimental import pallas as pl
from jax.experimental.pallas import tpu as pltpu
from jax.experimental.pallas import tpu_sc as plsc

H = 1024
F = 3584
E = 8
TOPK = 2
T = 2048
S = T * TOPK
BT = 256
NB = S // BT + E
NBT = NB * BT
FT = 896
NF = F // FT

NC = 2
NS = 16
NW = NC * NS

_SC_MESH = plsc.VectorSubcoreMesh(core_axis_name="c", subcore_axis_name="s")


TB = 256


def _router_body(x_ref, gw_ref, logits_ref, idx_ref, w0_ref, w1_ref):
    x = x_ref[...]
    gw = gw_ref[...]
    logits = lax.dot_general(x, gw, (((1,), (1,)), ((), ())),
                             preferred_element_type=jnp.float32)
    logits_ref[...] = logits
    lane = lax.broadcasted_iota(jnp.int32, (TB, E), 1)
    m1 = jnp.max(logits, axis=1, keepdims=True)
    i1 = jnp.min(jnp.where(logits == m1, lane, E), axis=1, keepdims=True)
    masked = jnp.where(lane == i1, -jnp.float32(1e30), logits)
    m2 = jnp.max(masked, axis=1, keepdims=True)
    i2 = jnp.min(jnp.where(masked == m2, lane, E), axis=1, keepdims=True)
    p1 = 1.0 / (1.0 + jnp.exp(m2 - m1))
    idx_ref[...] = jnp.concatenate([i1, i2], axis=1)
    w0_ref[...] = jnp.broadcast_to(p1, (TB, 16))
    w1_ref[...] = jnp.broadcast_to(1.0 - p1, (TB, 16))


def _router(x, gate_w):
    return pl.pallas_call(
        _router_body,
        grid=(T // TB,),
        in_specs=[
            pl.BlockSpec((TB, H), lambda i: (i, 0)),
            pl.BlockSpec((E, H), lambda i: (0, 0)),
        ],
        out_specs=[
            pl.BlockSpec((TB, E), lambda i: (i, 0)),
            pl.BlockSpec((TB, TOPK), lambda i: (i, 0)),
            pl.BlockSpec((TB, 16), lambda i: (i, 0)),
            pl.BlockSpec((TB, 16), lambda i: (i, 0)),
        ],
        out_shape=[
            jax.ShapeDtypeStruct((T, E), jnp.float32),
            jax.ShapeDtypeStruct((T, TOPK), jnp.int32),
            jax.ShapeDtypeStruct((T, 16), jnp.float32),
            jax.ShapeDtypeStruct((T, 16), jnp.float32),
        ],
    )(x, gate_w)


TPW = T // NW
DCH = 32
DNCH = TPW // DCH


@functools.partial(
    pl.kernel,
    mesh=_SC_MESH,
    out_type=jax.ShapeDtypeStruct((NBT, H), jnp.float32),
    scratch_types=[
        pltpu.VMEM((DCH,), jnp.int32),
        pltpu.VMEM((DCH,), jnp.int32),
        pltpu.VMEM((DCH,), jnp.int32),
        pltpu.VMEM((DCH,), jnp.int32),
        pltpu.VMEM((DCH, H), jnp.float32),
        pltpu.VMEM((DCH, H), jnp.float32),
        pltpu.SemaphoreType.DMA,
        pltpu.SemaphoreType.DMA,
        pltpu.SemaphoreType.DMA,
        pltpu.SemaphoreType.DMA,
        pltpu.SemaphoreType.DMA,
        pltpu.SemaphoreType.DMA,
    ],
)
def _dispatch(x_hbm, pos0_hbm, pos1_hbm, xs_hbm, p0a, p1a, p0b, p1b,
              bufa, bufb, sra, srb, sw0a, sw1a, sw0b, sw1b):
    wid = lax.axis_index("s") * NC + lax.axis_index("c")
    base = wid * TPW
    p0s = (p0a, p0b)
    p1s = (p1a, p1b)
    bufs = (bufa, bufb)
    srs = (sra, srb)
    sw0s = (sw0a, sw0b)
    sw1s = (sw1a, sw1b)
    reads = []
    for k in range(DNCH):
        off = base + k * DCH
        pltpu.sync_copy(pos0_hbm.at[pl.ds(off, DCH)], p0s[k])
        pltpu.sync_copy(pos1_hbm.at[pl.ds(off, DCH)], p1s[k])
        reads.append(
            pltpu.async_copy(x_hbm.at[pl.ds(off, DCH)], bufs[k], srs[k]))
    writes = []
    for k in range(DNCH):
        reads[k].wait()
        writes.append(
            pltpu.async_copy(bufs[k], xs_hbm.at[p0s[k]], sw0s[k]))
        writes.append(
            pltpu.async_copy(bufs[k], xs_hbm.at[p1s[k]], sw1s[k]))
    for w in writes:
        w.wait()


def _cast_body(xs_ref, xb_ref):
    xb_ref[...] = xs_ref[...].astype(jnp.bfloat16)


def _cast_bf16(xs):
    cb = 128
    return pl.pallas_call(
        _cast_body,
        grid=(NBT // cb,),
        in_specs=[pl.BlockSpec((cb, H), lambda b: (b, 0))],
        out_specs=pl.BlockSpec((cb, H), lambda b: (b, 0)),
        out_shape=jax.ShapeDtypeStruct((NBT, H), jnp.bfloat16),
    )(xs)


def _ffn_body(seid_ref, sval_ref, xs_ref, w1_ref, w3_ref, w2_ref, y_ref,
              w1bf, w3bf, w2bf, yacc):
    f = pl.program_id(0)
    b = pl.program_id(1)
    is_new = jnp.logical_or(
        b == 0, seid_ref[b] != seid_ref[jnp.maximum(b - 1, 0)])

    @pl.when(is_new)
    def _():
        w1bf[...] = w1_ref[0].astype(jnp.bfloat16)
        w3bf[...] = w3_ref[0].astype(jnp.bfloat16)
        w2bf[...] = w2_ref[0].astype(jnp.bfloat16)

    @pl.when(sval_ref[b] > 0)
    def _():
        x = xs_ref[...]
        g = lax.dot_general(x, w1bf[...], (((1,), (1,)), ((), ())),
                            preferred_element_type=jnp.float32)
        u = lax.dot_general(x, w3bf[...], (((1,), (1,)), ((), ())),
                            preferred_element_type=jnp.float32)
        h = ((g * jax.nn.sigmoid(g)) * u).astype(jnp.bfloat16)
        part = lax.dot_general(h, w2bf[...], (((1,), (1,)), ((), ())),
                               preferred_element_type=jnp.float32)
        sl = pl.ds(b * BT, BT)

        @pl.when(f == 0)
        def _():
            yacc[sl, :] = part

        @pl.when(jnp.logical_and(f > 0, f < NF - 1))
        def _():
            yacc[sl, :] += part

        @pl.when(f == NF - 1)
        def _():
            y_ref[...] = yacc[sl, :] + part


def _ffn(beid, bval, xs, w1, w3, w2):
    grid_spec = pltpu.PrefetchScalarGridSpec(
        num_scalar_prefetch=2,
        grid=(NF, NB),
        in_specs=[
            pl.BlockSpec((BT, H), lambda f, b, seid, sval: (b, 0)),
            pl.BlockSpec((1, FT, H), lambda f, b, seid, sval: (seid[b], f, 0)),
            pl.BlockSpec((1, FT, H), lambda f, b, seid, sval: (seid[b], f, 0)),
            pl.BlockSpec((1, H, FT), lambda f, b, seid, sval: (seid[b], 0, f)),
        ],
        out_specs=pl.BlockSpec(
            (BT, H),
            lambda f, b, seid, sval: (jnp.where(f == NF - 1, b, 0), 0)),
        scratch_shapes=[
            pltpu.VMEM((FT, H), jnp.bfloat16),
            pltpu.VMEM((FT, H), jnp.bfloat16),
            pltpu.VMEM((H, FT), jnp.bfloat16),
            pltpu.VMEM((NBT, H), jnp.float32),
        ],
    )
    return pl.pallas_call(
        _ffn_body,
        grid_spec=grid_spec,
        out_shape=jax.ShapeDtypeStruct((NBT, H), jnp.float32),
        compiler_params=pltpu.CompilerParams(
            dimension_semantics=("arbitrary", "arbitrary")),
    )(beid, bval, xs, w1, w3, w2)


CCH = 16
CNCH = TPW // CCH


@functools.partial(
    pl.kernel,
    mesh=_SC_MESH,
    out_type=jax.ShapeDtypeStruct((T, H), jnp.float32),
    scratch_types=[
        pltpu.VMEM((CCH,), jnp.int32),
        pltpu.VMEM((CCH,), jnp.int32),
        pltpu.VMEM((CCH,), jnp.int32),
        pltpu.VMEM((CCH,), jnp.int32),
        pltpu.VMEM((CCH, 16), jnp.float32),
        pltpu.VMEM((CCH, 16), jnp.float32),
        pltpu.VMEM((CCH, 16), jnp.float32),
        pltpu.VMEM((CCH, 16), jnp.float32),
        pltpu.VMEM((CCH, H), jnp.float32),
        pltpu.VMEM((CCH, H), jnp.float32),
        pltpu.VMEM((CCH, H), jnp.float32),
        pltpu.VMEM((CCH, H), jnp.float32),
        pltpu.SemaphoreType.DMA,
        pltpu.SemaphoreType.DMA,
        pltpu.SemaphoreType.DMA,
        pltpu.SemaphoreType.DMA,
    ],
)
def _combine(y_hbm, pos0_hbm, pos1_hbm, wt0_hbm, wt1_hbm, out_hbm,
             ia0, ib0, ia1, ib1, wa0, wb0, wa1, wb1,
             a0, b0, a1, b1, sa0, sb0, sa1, sb1):
    wid = lax.axis_index("s") * NC + lax.axis_index("c")
    base = wid * TPW
    iabufs = (ia0, ia1)
    ibbufs = (ib0, ib1)
    wabufs = (wa0, wa1)
    wbbufs = (wb0, wb1)
    abufs = (a0, a1)
    bbufs = (b0, b1)
    sas = (sa0, sa1)
    sbs = (sb0, sb1)

    def start(c, k):
        off = base + c * CCH
        pltpu.sync_copy(pos0_hbm.at[pl.ds(off, CCH)], iabufs[k])
        pltpu.sync_copy(pos1_hbm.at[pl.ds(off, CCH)], ibbufs[k])
        pltpu.sync_copy(wt0_hbm.at[pl.ds(off, CCH)], wabufs[k])
        pltpu.sync_copy(wt1_hbm.at[pl.ds(off, CCH)], wbbufs[k])
        return (pltpu.async_copy(y_hbm.at[iabufs[k]], abufs[k], sas[k]),
                pltpu.async_copy(y_hbm.at[ibbufs[k]], bbufs[k], sbs[k]))

    cps = [start(0, 0), start(1, 1)]
    for c in range(CNCH):
        k = c % 2
        cps[k][0].wait()
        cps[k][1].wait()
        av, bv = abufs[k], bbufs[k]
        wav, wbv = wabufs[k], wbbufs[k]

        def row(r, rc):
            w0r = wav[r, :]
            w1r = wbv[r, :]
            for j in range(H // 16):
                sl = pl.ds(j * 16, 16)
                av[r, sl] = av[r, sl] * w0r + bv[r, sl] * w1r
            return rc

        lax.fori_loop(0, CCH, row, 0)
        pltpu.sync_copy(av, out_hbm.at[pl.ds(base + c * CCH, CCH)])
        if c + 2 < CNCH:
            cps[k] = start(c + 2, k)


def kernel(hidden_states, gate_w, w1, w3, w2):
    orig_shape = hidden_states.shape
    x = hidden_states.reshape(T, H)

    logits, top_idx, w16a, w16b = _router(x, gate_w)

    eid = top_idx.reshape(S)
    onehot = (eid[:, None] == jnp.arange(E, dtype=jnp.int32)[None, :])
    onehot = onehot.astype(jnp.int32)
    ranks = jnp.cumsum(onehot, axis=0) - onehot
    rank = jnp.take_along_axis(ranks, eid[:, None], axis=1)[:, 0]
    counts = jnp.sum(onehot, axis=0)
    padded = ((counts + BT - 1) // BT) * BT
    astart = jnp.concatenate(
        [jnp.zeros((1,), jnp.int32), jnp.cumsum(padded)[:-1]])
    pos = astart[eid] + rank
    bstart = jnp.arange(NB, dtype=jnp.int32) * BT
    gend = astart + padded
    beid = jnp.minimum(
        jnp.sum((bstart[:, None] >= gend[None, :]).astype(jnp.int32), axis=1),
        E - 1)
    bval = (bstart < (astart + counts)[beid]).astype(jnp.int32)
    pos2 = pos.reshape(T, TOPK)
    pos0 = pos2[:, 0]
    pos1 = pos2[:, 1]

    xs = _dispatch(x, pos0, pos1)
    y = _ffn(beid, bval, _cast_bf16(xs), w1, w3, w2)
    out = _combine(y, pos0, pos1, w16a, w16b)
    return (out.reshape(orig_shape), logits)


# speedup vs baseline: 1.9810x; 1.0022x over previous
import functools

import jax
import jax.numpy as jnp
from jax import lax
from jax.experimental import pallas as pl
from jax.experimental.pallas import tpu as pltpu
from jax.experimental.pallas import tpu_sc as plsc

H = 1024
F = 3584
E = 8
TOPK = 2
T = 2048
S = T * TOPK
BT = 256
NB = S // BT + E
NBT = NB * BT
FT = 896
NF = F // FT

NC = 2
NS = 16
NW = NC * NS

_SC_MESH = plsc.VectorSubcoreMesh(core_axis_name="c", subcore_axis_name="s")


TB = 256


def _router_body(x_ref, gw_ref, logits_ref, idx_ref, w0_ref, w1_ref):
    x = x_ref[...]
    gw = gw_ref[...]
    logits = lax.dot_general(x, gw, (((1,), (1,)), ((), ())),
                             preferred_element_type=jnp.float32)
    logits_ref[...] = logits
    lane = lax.broadcasted_iota(jnp.int32, (TB, E), 1)
    m1 = jnp.max(logits, axis=1, keepdims=True)
    i1 = jnp.min(jnp.where(logits == m1, lane, E), axis=1, keepdims=True)
    masked = jnp.where(lane == i1, -jnp.float32(1e30), logits)
    m2 = jnp.max(masked, axis=1, keepdims=True)
    i2 = jnp.min(jnp.where(masked == m2, lane, E), axis=1, keepdims=True)
    p1 = 1.0 / (1.0 + jnp.exp(m2 - m1))
    idx_ref[...] = jnp.concatenate([i1, i2], axis=1)
    w0_ref[...] = jnp.broadcast_to(p1, (TB, 16))
    w1_ref[...] = jnp.broadcast_to(1.0 - p1, (TB, 16))


def _router(x, gate_w):
    return pl.pallas_call(
        _router_body,
        grid=(T // TB,),
        in_specs=[
            pl.BlockSpec((TB, H), lambda i: (i, 0)),
            pl.BlockSpec((E, H), lambda i: (0, 0)),
        ],
        out_specs=[
            pl.BlockSpec((TB, E), lambda i: (i, 0)),
            pl.BlockSpec((TB, TOPK), lambda i: (i, 0)),
            pl.BlockSpec((TB, 16), lambda i: (i, 0)),
            pl.BlockSpec((TB, 16), lambda i: (i, 0)),
        ],
        out_shape=[
            jax.ShapeDtypeStruct((T, E), jnp.float32),
            jax.ShapeDtypeStruct((T, TOPK), jnp.int32),
            jax.ShapeDtypeStruct((T, 16), jnp.float32),
            jax.ShapeDtypeStruct((T, 16), jnp.float32),
        ],
    )(x, gate_w)


TPW = T // NW
DCH = 32
DNCH = TPW // DCH


@functools.partial(
    pl.kernel,
    mesh=_SC_MESH,
    out_type=jax.ShapeDtypeStruct((NBT, H), jnp.float32),
    scratch_types=[
        pltpu.VMEM((DCH,), jnp.int32),
        pltpu.VMEM((DCH,), jnp.int32),
        pltpu.VMEM((DCH,), jnp.int32),
        pltpu.VMEM((DCH,), jnp.int32),
        pltpu.VMEM((DCH, H), jnp.float32),
        pltpu.VMEM((DCH, H), jnp.float32),
        pltpu.SemaphoreType.DMA,
        pltpu.SemaphoreType.DMA,
        pltpu.SemaphoreType.DMA,
        pltpu.SemaphoreType.DMA,
        pltpu.SemaphoreType.DMA,
        pltpu.SemaphoreType.DMA,
    ],
)
def _dispatch(x_hbm, pos0_hbm, pos1_hbm, xs_hbm, p0a, p1a, p0b, p1b,
              bufa, bufb, sra, srb, sw0a, sw1a, sw0b, sw1b):
    wid = lax.axis_index("s") * NC + lax.axis_index("c")
    base = wid * TPW
    p0s = (p0a, p0b)
    p1s = (p1a, p1b)
    bufs = (bufa, bufb)
    srs = (sra, srb)
    sw0s = (sw0a, sw0b)
    sw1s = (sw1a, sw1b)
    reads = []
    for k in range(DNCH):
        off = base + k * DCH
        pltpu.sync_copy(pos0_hbm.at[pl.ds(off, DCH)], p0s[k])
        pltpu.sync_copy(pos1_hbm.at[pl.ds(off, DCH)], p1s[k])
        reads.append(
            pltpu.async_copy(x_hbm.at[pl.ds(off, DCH)], bufs[k], srs[k]))
    writes = []
    for k in range(DNCH):
        reads[k].wait()
        writes.append(
            pltpu.async_copy(bufs[k], xs_hbm.at[p0s[k]], sw0s[k]))
        writes.append(
            pltpu.async_copy(bufs[k], xs_hbm.at[p1s[k]], sw1s[k]))
    for w in writes:
        w.wait()


def _cast_body(xs_ref, xb_ref):
    xb_ref[...] = xs_ref[...].astype(jnp.bfloat16)


def _cast_bf16(xs):
    cb = 128
    return pl.pallas_call(
        _cast_body,
        grid=(NBT // cb,),
        in_specs=[pl.BlockSpec((cb, H), lambda b: (b, 0))],
        out_specs=pl.BlockSpec((cb, H), lambda b: (b, 0)),
        out_shape=jax.ShapeDtypeStruct((NBT, H), jnp.bfloat16),
    )(xs)


def _ffn_body(seid_ref, sval_ref, xs_ref, w1_ref, w3_ref, w2_ref, y_ref,
              w1bf, w3bf, w2bf, yacc):
    f = pl.program_id(0)
    b = pl.program_id(1)
    is_new = jnp.logical_or(
        b == 0, seid_ref[b] != seid_ref[jnp.maximum(b - 1, 0)])

    @pl.when(is_new)
    def _():
        w1bf[...] = w1_ref[0].astype(jnp.bfloat16)
        w3bf[...] = w3_ref[0].astype(jnp.bfloat16)
        w2bf[...] = w2_ref[0].astype(jnp.bfloat16)

    @pl.when(sval_ref[b] > 0)
    def _():
        x = xs_ref[...]
        g = lax.dot_general(x, w1bf[...], (((1,), (1,)), ((), ())),
                            preferred_element_type=jnp.float32)
        u = lax.dot_general(x, w3bf[...], (((1,), (1,)), ((), ())),
                            preferred_element_type=jnp.float32)
        h = ((g * jax.nn.sigmoid(g)) * u).astype(jnp.bfloat16)
        part = lax.dot_general(h, w2bf[...], (((1,), (1,)), ((), ())),
                               preferred_element_type=jnp.float32)
        sl = pl.ds(b * BT, BT)

        @pl.when(f == 0)
        def _():
            yacc[sl, :] = part

        @pl.when(jnp.logical_and(f > 0, f < NF - 1))
        def _():
            yacc[sl, :] += part

        @pl.when(f == NF - 1)
        def _():
            y_ref[...] = yacc[sl, :] + part


def _ffn(beid, bval, xs, w1, w3, w2):
    grid_spec = pltpu.PrefetchScalarGridSpec(
        num_scalar_prefetch=2,
        grid=(NF, NB),
        in_specs=[
            pl.BlockSpec((BT, H), lambda f, b, seid, sval: (b, 0)),
            pl.BlockSpec((1, FT, H), lambda f, b, seid, sval: (seid[b], f, 0)),
            pl.BlockSpec((1, FT, H), lambda f, b, seid, sval: (seid[b], f, 0)),
            pl.BlockSpec((1, H, FT), lambda f, b, seid, sval: (seid[b], 0, f)),
        ],
        out_specs=pl.BlockSpec(
            (BT, H),
            lambda f, b, seid, sval: (jnp.where(f == NF - 1, b, 0), 0)),
        scratch_shapes=[
            pltpu.VMEM((FT, H), jnp.bfloat16),
            pltpu.VMEM((FT, H), jnp.bfloat16),
            pltpu.VMEM((H, FT), jnp.bfloat16),
            pltpu.VMEM((NBT, H), jnp.float32),
        ],
    )
    return pl.pallas_call(
        _ffn_body,
        grid_spec=grid_spec,
        out_shape=jax.ShapeDtypeStruct((NBT, H), jnp.float32),
        compiler_params=pltpu.CompilerParams(
            dimension_semantics=("arbitrary", "arbitrary")),
    )(beid, bval, xs, w1, w3, w2)


CCH = 16
CNCH = TPW // CCH


@functools.partial(
    pl.kernel,
    mesh=_SC_MESH,
    out_type=jax.ShapeDtypeStruct((T, H), jnp.float32),
    scratch_types=[
        pltpu.VMEM((CCH,), jnp.int32),
        pltpu.VMEM((CCH,), jnp.int32),
        pltpu.VMEM((CCH,), jnp.int32),
        pltpu.VMEM((CCH,), jnp.int32),
        pltpu.VMEM((CCH, 16), jnp.float32),
        pltpu.VMEM((CCH, 16), jnp.float32),
        pltpu.VMEM((CCH, 16), jnp.float32),
        pltpu.VMEM((CCH, 16), jnp.float32),
        pltpu.VMEM((CCH, H), jnp.float32),
        pltpu.VMEM((CCH, H), jnp.float32),
        pltpu.VMEM((CCH, H), jnp.float32),
        pltpu.VMEM((CCH, H), jnp.float32),
        pltpu.SemaphoreType.DMA,
        pltpu.SemaphoreType.DMA,
        pltpu.SemaphoreType.DMA,
        pltpu.SemaphoreType.DMA,
    ],
)
def _combine(y_hbm, pos0_hbm, pos1_hbm, wt0_hbm, wt1_hbm, out_hbm,
             ia0, ib0, ia1, ib1, wa0, wb0, wa1, wb1,
             a0, b0, a1, b1, sa0, sb0, sa1, sb1):
    wid = lax.axis_index("s") * NC + lax.axis_index("c")
    base = wid * TPW
    iabufs = (ia0, ia1)
    ibbufs = (ib0, ib1)
    wabufs = (wa0, wa1)
    wbbufs = (wb0, wb1)
    abufs = (a0, a1)
    bbufs = (b0, b1)
    sas = (sa0, sa1)
    sbs = (sb0, sb1)

    def start(c, k):
        off = base + c * CCH
        pltpu.sync_copy(pos0_hbm.at[pl.ds(off, CCH)], iabufs[k])
        pltpu.sync_copy(pos1_hbm.at[pl.ds(off, CCH)], ibbufs[k])
        pltpu.sync_copy(wt0_hbm.at[pl.ds(off, CCH)], wabufs[k])
        pltpu.sync_copy(wt1_hbm.at[pl.ds(off, CCH)], wbbufs[k])
        return (pltpu.async_copy(y_hbm.at[iabufs[k]], abufs[k], sas[k]),
                pltpu.async_copy(y_hbm.at[ibbufs[k]], bbufs[k], sbs[k]))

    cps = [start(0, 0), start(1, 1)]
    for c in range(CNCH):
        k = c % 2
        cps[k][0].wait()
        cps[k][1].wait()
        av, bv = abufs[k], bbufs[k]
        wav, wbv = wabufs[k], wbbufs[k]

        def row(r, rc):
            w0r = wav[r, :]
            w1r = wbv[r, :]
            for j in range(H // 16):
                sl = pl.ds(j * 16, 16)
                av[r, sl] = av[r, sl] * w0r + bv[r, sl] * w1r
            return rc

        lax.fori_loop(0, CCH, row, 0)
        pltpu.sync_copy(av, out_hbm.at[pl.ds(base + c * CCH, CCH)])
        if c + 2 < CNCH:
            cps[k] = start(c + 2, k)


def kernel(hidden_states, gate_w, w1, w3, w2):
    orig_shape = hidden_states.shape
    x = hidden_states.reshape(T, H)

    logits, top_idx, w16a, w16b = _router(x, gate_w)

    eid = top_idx.reshape(S)
    onehot = (eid[:, None] == jnp.arange(E, dtype=jnp.int32)[None, :])
    onehot = onehot.astype(jnp.int32)
    ranks = jnp.cumsum(onehot, axis=0) - onehot
    rank = jnp.take_along_axis(ranks, eid[:, None], axis=1)[:, 0]
    counts = jnp.sum(onehot, axis=0)
    padded = ((counts + BT - 1) // BT) * BT
    astart = jnp.concatenate(
        [jnp.zeros((1,), jnp.int32), jnp.cumsum(padded)[:-1]])
    pos = (astart + padded - counts)[eid] + rank
    bstart = jnp.arange(NB, dtype=jnp.int32) * BT
    gend = astart + padded
    beid = jnp.minimum(
        jnp.sum((bstart[:, None] >= gend[None, :]).astype(jnp.int32), axis=1),
        E - 1)
    bval = jnp.logical_and(
        bstart + BT > (astart + padded - counts)[beid],
        bstart < gend[beid]).astype(jnp.int32)
    pos2 = pos.reshape(T, TOPK)
    pos0 = pos2[:, 0]
    pos1 = pos2[:, 1]

    xs = _dispatch(x, pos0, pos1)
    y = _ffn(beid, bval, _cast_bf16(xs), w1, w3, w2)
    out = _combine(y, pos0, pos1, w16a, w16b)
    return (out.reshape(orig_shape), logits)


# speedup vs baseline: 2.0278x; 1.0236x over previous
import functools

import jax
import jax.numpy as jnp
from jax import lax
from jax.experimental import pallas as pl
from jax.experimental.pallas import tpu as pltpu
from jax.experimental.pallas import tpu_sc as plsc

H = 1024
F = 3584
E = 8
TOPK = 2
T = 2048
S = T * TOPK
BT = 256
NB = S // BT + E
NBT = NB * BT
FT = 896
NF = F // FT

NC = 2
NS = 16
NW = NC * NS

_SC_MESH = plsc.VectorSubcoreMesh(core_axis_name="c", subcore_axis_name="s")


TB = 256


def _router_body(x_ref, gw_ref, logits_ref, idx_ref, w0_ref, w1_ref):
    x = x_ref[...]
    gw = gw_ref[...]
    logits = lax.dot_general(x, gw, (((1,), (1,)), ((), ())),
                             preferred_element_type=jnp.float32)
    logits_ref[...] = logits
    lane = lax.broadcasted_iota(jnp.int32, (TB, E), 1)
    m1 = jnp.max(logits, axis=1, keepdims=True)
    i1 = jnp.min(jnp.where(logits == m1, lane, E), axis=1, keepdims=True)
    masked = jnp.where(lane == i1, -jnp.float32(1e30), logits)
    m2 = jnp.max(masked, axis=1, keepdims=True)
    i2 = jnp.min(jnp.where(masked == m2, lane, E), axis=1, keepdims=True)
    p1 = 1.0 / (1.0 + jnp.exp(m2 - m1))
    idx_ref[...] = jnp.concatenate([i1, i2], axis=1)
    w0_ref[...] = jnp.broadcast_to(p1, (TB, 16))
    w1_ref[...] = jnp.broadcast_to(1.0 - p1, (TB, 16))


def _router(x, gate_w):
    return pl.pallas_call(
        _router_body,
        grid=(T // TB,),
        in_specs=[
            pl.BlockSpec((TB, H), lambda i: (i, 0)),
            pl.BlockSpec((E, H), lambda i: (0, 0)),
        ],
        out_specs=[
            pl.BlockSpec((TB, E), lambda i: (i, 0)),
            pl.BlockSpec((TB, TOPK), lambda i: (i, 0)),
            pl.BlockSpec((TB, 16), lambda i: (i, 0)),
            pl.BlockSpec((TB, 16), lambda i: (i, 0)),
        ],
        out_shape=[
            jax.ShapeDtypeStruct((T, E), jnp.float32),
            jax.ShapeDtypeStruct((T, TOPK), jnp.int32),
            jax.ShapeDtypeStruct((T, 16), jnp.float32),
            jax.ShapeDtypeStruct((T, 16), jnp.float32),
        ],
    )(x, gate_w)


TPW = T // NW
DCH = 32
DNCH = TPW // DCH


@functools.partial(
    pl.kernel,
    mesh=_SC_MESH,
    out_type=jax.ShapeDtypeStruct((NBT, H), jnp.float32),
    scratch_types=[
        pltpu.VMEM((DCH,), jnp.int32),
        pltpu.VMEM((DCH,), jnp.int32),
        pltpu.VMEM((DCH,), jnp.int32),
        pltpu.VMEM((DCH,), jnp.int32),
        pltpu.VMEM((DCH, H), jnp.float32),
        pltpu.VMEM((DCH, H), jnp.float32),
        pltpu.SemaphoreType.DMA,
        pltpu.SemaphoreType.DMA,
        pltpu.SemaphoreType.DMA,
        pltpu.SemaphoreType.DMA,
        pltpu.SemaphoreType.DMA,
        pltpu.SemaphoreType.DMA,
    ],
)
def _dispatch(x_hbm, pos0_hbm, pos1_hbm, xs_hbm, p0a, p1a, p0b, p1b,
              bufa, bufb, sra, srb, sw0a, sw1a, sw0b, sw1b):
    wid = lax.axis_index("s") * NC + lax.axis_index("c")
    base = wid * TPW
    p0s = (p0a, p0b)
    p1s = (p1a, p1b)
    bufs = (bufa, bufb)
    srs = (sra, srb)
    sw0s = (sw0a, sw0b)
    sw1s = (sw1a, sw1b)
    reads = []
    for k in range(DNCH):
        off = base + k * DCH
        pltpu.sync_copy(pos0_hbm.at[pl.ds(off, DCH)], p0s[k])
        pltpu.sync_copy(pos1_hbm.at[pl.ds(off, DCH)], p1s[k])
        reads.append(
            pltpu.async_copy(x_hbm.at[pl.ds(off, DCH)], bufs[k], srs[k]))
    writes = []
    for k in range(DNCH):
        reads[k].wait()
        writes.append(
            pltpu.async_copy(bufs[k], xs_hbm.at[p0s[k]], sw0s[k]))
        writes.append(
            pltpu.async_copy(bufs[k], xs_hbm.at[p1s[k]], sw1s[k]))
    for w in writes:
        w.wait()


def _cast_body(xs_ref, xb_ref):
    xb_ref[...] = xs_ref[...].astype(jnp.bfloat16)


def _cast_bf16(xs):
    cb = 256
    return pl.pallas_call(
        _cast_body,
        grid=(NBT // cb,),
        in_specs=[pl.BlockSpec((cb, H), lambda b: (b, 0))],
        out_specs=pl.BlockSpec((cb, H), lambda b: (b, 0)),
        out_shape=jax.ShapeDtypeStruct((NBT, H), jnp.bfloat16),
    )(xs)


def _ffn_body(seid_ref, sval_ref, xs_ref, w1_ref, w3_ref, w2_ref, y_ref,
              w1bf, w3bf, w2bf, yacc):
    f = pl.program_id(0)
    b = pl.program_id(1)
    is_new = jnp.logical_or(
        b == 0, seid_ref[b] != seid_ref[jnp.maximum(b - 1, 0)])

    @pl.when(is_new)
    def _():
        w1bf[...] = w1_ref[0].astype(jnp.bfloat16)
        w3bf[...] = w3_ref[0].astype(jnp.bfloat16)
        w2bf[...] = w2_ref[0].astype(jnp.bfloat16)

    @pl.when(sval_ref[b] > 0)
    def _():
        x = xs_ref[...]
        g = lax.dot_general(x, w1bf[...], (((1,), (1,)), ((), ())),
                            preferred_element_type=jnp.float32)
        u = lax.dot_general(x, w3bf[...], (((1,), (1,)), ((), ())),
                            preferred_element_type=jnp.float32)
        h = ((g * jax.nn.sigmoid(g)) * u).astype(jnp.bfloat16)
        part = lax.dot_general(h, w2bf[...], (((1,), (1,)), ((), ())),
                               preferred_element_type=jnp.float32)
        sl = pl.ds(b * BT, BT)

        @pl.when(f == 0)
        def _():
            yacc[sl, :] = part

        @pl.when(jnp.logical_and(f > 0, f < NF - 1))
        def _():
            yacc[sl, :] += part

        @pl.when(f == NF - 1)
        def _():
            y_ref[...] = yacc[sl, :] + part


def _ffn(beid, bval, xs, w1, w3, w2):
    grid_spec = pltpu.PrefetchScalarGridSpec(
        num_scalar_prefetch=2,
        grid=(NF, NB),
        in_specs=[
            pl.BlockSpec((BT, H), lambda f, b, seid, sval: (b, 0)),
            pl.BlockSpec((1, FT, H), lambda f, b, seid, sval: (seid[b], f, 0)),
            pl.BlockSpec((1, FT, H), lambda f, b, seid, sval: (seid[b], f, 0)),
            pl.BlockSpec((1, H, FT), lambda f, b, seid, sval: (seid[b], 0, f)),
        ],
        out_specs=pl.BlockSpec(
            (BT, H),
            lambda f, b, seid, sval: (jnp.where(f == NF - 1, b, 0), 0)),
        scratch_shapes=[
            pltpu.VMEM((FT, H), jnp.bfloat16),
            pltpu.VMEM((FT, H), jnp.bfloat16),
            pltpu.VMEM((H, FT), jnp.bfloat16),
            pltpu.VMEM((NBT, H), jnp.float32),
        ],
    )
    return pl.pallas_call(
        _ffn_body,
        grid_spec=grid_spec,
        out_shape=jax.ShapeDtypeStruct((NBT, H), jnp.float32),
        compiler_params=pltpu.CompilerParams(
            dimension_semantics=("arbitrary", "arbitrary")),
    )(beid, bval, xs, w1, w3, w2)


CCH = 16
CNCH = TPW // CCH


@functools.partial(
    pl.kernel,
    mesh=_SC_MESH,
    out_type=jax.ShapeDtypeStruct((T, H), jnp.float32),
    scratch_types=[
        pltpu.VMEM((CCH,), jnp.int32),
        pltpu.VMEM((CCH,), jnp.int32),
        pltpu.VMEM((CCH,), jnp.int32),
        pltpu.VMEM((CCH,), jnp.int32),
        pltpu.VMEM((CCH, 16), jnp.float32),
        pltpu.VMEM((CCH, 16), jnp.float32),
        pltpu.VMEM((CCH, 16), jnp.float32),
        pltpu.VMEM((CCH, 16), jnp.float32),
        pltpu.VMEM((CCH, H), jnp.float32),
        pltpu.VMEM((CCH, H), jnp.float32),
        pltpu.VMEM((CCH, H), jnp.float32),
        pltpu.VMEM((CCH, H), jnp.float32),
        pltpu.SemaphoreType.DMA,
        pltpu.SemaphoreType.DMA,
        pltpu.SemaphoreType.DMA,
        pltpu.SemaphoreType.DMA,
    ],
)
def _combine(y_hbm, pos0_hbm, pos1_hbm, wt0_hbm, wt1_hbm, out_hbm,
             ia0, ib0, ia1, ib1, wa0, wb0, wa1, wb1,
             a0, b0, a1, b1, sa0, sb0, sa1, sb1):
    wid = lax.axis_index("s") * NC + lax.axis_index("c")
    base = wid * TPW
    iabufs = (ia0, ia1)
    ibbufs = (ib0, ib1)
    wabufs = (wa0, wa1)
    wbbufs = (wb0, wb1)
    abufs = (a0, a1)
    bbufs = (b0, b1)
    sas = (sa0, sa1)
    sbs = (sb0, sb1)

    def start(c, k):
        off = base + c * CCH
        pltpu.sync_copy(pos0_hbm.at[pl.ds(off, CCH)], iabufs[k])
        pltpu.sync_copy(pos1_hbm.at[pl.ds(off, CCH)], ibbufs[k])
        pltpu.sync_copy(wt0_hbm.at[pl.ds(off, CCH)], wabufs[k])
        pltpu.sync_copy(wt1_hbm.at[pl.ds(off, CCH)], wbbufs[k])
        return (pltpu.async_copy(y_hbm.at[iabufs[k]], abufs[k], sas[k]),
                pltpu.async_copy(y_hbm.at[ibbufs[k]], bbufs[k], sbs[k]))

    cps = [start(0, 0), start(1, 1)]
    for c in range(CNCH):
        k = c % 2
        cps[k][0].wait()
        cps[k][1].wait()
        av, bv = abufs[k], bbufs[k]
        wav, wbv = wabufs[k], wbbufs[k]

        def row(r, rc):
            w0r = wav[r, :]
            w1r = wbv[r, :]
            for j in range(H // 16):
                sl = pl.ds(j * 16, 16)
                av[r, sl] = av[r, sl] * w0r + bv[r, sl] * w1r
            return rc

        lax.fori_loop(0, CCH, row, 0)
        pltpu.sync_copy(av, out_hbm.at[pl.ds(base + c * CCH, CCH)])
        if c + 2 < CNCH:
            cps[k] = start(c + 2, k)


def kernel(hidden_states, gate_w, w1, w3, w2):
    orig_shape = hidden_states.shape
    x = hidden_states.reshape(T, H)

    logits, top_idx, w16a, w16b = _router(x, gate_w)

    eid = top_idx.reshape(S)
    onehot = (eid[:, None] == jnp.arange(E, dtype=jnp.int32)[None, :])
    onehot = onehot.astype(jnp.int32)
    ranks = jnp.cumsum(onehot, axis=0) - onehot
    rank = jnp.take_along_axis(ranks, eid[:, None], axis=1)[:, 0]
    counts = jnp.sum(onehot, axis=0)
    padded = ((counts + BT - 1) // BT) * BT
    astart = jnp.concatenate(
        [jnp.zeros((1,), jnp.int32), jnp.cumsum(padded)[:-1]])
    pos = (astart + padded - counts)[eid] + rank
    bstart = jnp.arange(NB, dtype=jnp.int32) * BT
    gend = astart + padded
    beid = jnp.minimum(
        jnp.sum((bstart[:, None] >= gend[None, :]).astype(jnp.int32), axis=1),
        E - 1)
    bval = jnp.logical_and(
        bstart + BT > (astart + padded - counts)[beid],
        bstart < gend[beid]).astype(jnp.int32)
    pos2 = pos.reshape(T, TOPK)
    pos0 = pos2[:, 0]
    pos1 = pos2[:, 1]

    xs = _dispatch(x, pos0, pos1)
    y = _ffn(beid, bval, _cast_bf16(xs), w1, w3, w2)
    out = _combine(y, pos0, pos1, w16a, w16b)
    return (out.reshape(orig_shape), logits)
